# baseline (speedup 1.0000x reference)
import functools

import jax
import jax.numpy as jnp
from jax import lax
from jax.experimental import pallas as pl
from jax.experimental.pallas import tpu as pltpu
from jax.experimental.pallas import tpu_sc as plsc

_NUM_LABELS = 50
_WORD_DIM = 50
_POS_DIM = 15
_L64 = 64


def _proj_body(tabt_ref, W_ref, ptab_ref, b_ref, out_ref, pout_ref):
    xt = tabt_ref[...]
    W64 = W_ref[...]
    Ww = W64[:, :_WORD_DIM]
    scores = lax.dot_general(
        xt.astype(jnp.bfloat16), Ww.astype(jnp.bfloat16),
        (((0,), (1,)), ((), ())), preferred_element_type=jnp.float32)
    out_ref[...] = scores + b_ref[...]

    @pl.when(pl.program_id(0) == 0)
    def _():
        Wp = W64[:, _WORD_DIM:]
        pout_ref[...] = lax.dot_general(ptab_ref[...], Wp,
                                        (((1,), (1,)), ((), ())),
                                        precision=lax.Precision.HIGHEST)


def _project(table_t, pos_table, W64, b64, V):
    blk = 16384
    grid = (pl.cdiv(V, blk),)
    return pl.pallas_call(
        _proj_body,
        grid=grid,
        in_specs=[
            pl.BlockSpec((_WORD_DIM, blk), lambda i: (0, i)),
            pl.BlockSpec((_L64, _WORD_DIM + _POS_DIM), lambda i: (0, 0)),
            pl.BlockSpec((_NUM_LABELS, _POS_DIM), lambda i: (0, 0)),
            pl.BlockSpec((1, _L64), lambda i: (0, 0)),
        ],
        out_specs=[
            pl.BlockSpec((blk, _L64), lambda i: (i, 0)),
            pl.BlockSpec((_NUM_LABELS, _L64), lambda i: (0, 0)),
        ],
        out_shape=[
            jax.ShapeDtypeStruct((V, _L64), jnp.float32),
            jax.ShapeDtypeStruct((_NUM_LABELS, _L64), jnp.float32),
        ],
    )(table_t, W64, pos_table, b64.reshape(1, _L64))


def _sc_gather_add(p_tab, p_pos, idx, pos, B):
    info = plsc.get_sparse_core_info()
    nw = info.num_cores * info.num_subcores
    b_per_w = B // nw
    mesh = plsc.VectorSubcoreMesh(core_axis_name="c", subcore_axis_name="s")

    @functools.partial(
        pl.kernel,
        mesh=mesh,
        out_type=jax.ShapeDtypeStruct((B, _L64), jnp.float32),
        compiler_params=pltpu.CompilerParams(use_tc_tiling_on_sc=True),
        scratch_types=[
            pltpu.VMEM((b_per_w,), jnp.int32),
            pltpu.VMEM((b_per_w,), jnp.int32),
            pltpu.VMEM((b_per_w, _L64), jnp.float32),
            pltpu.VMEM((_NUM_LABELS, _L64), jnp.float32),
            pltpu.SemaphoreType.DMA,
        ],
    )
    def gather_k(ptab_hbm, ppos_hbm, idx_hbm, pos_hbm, out_hbm,
                 idx_v, pos_v, rows_v, ppos_v, sem):
        wid = lax.axis_index("s") * info.num_cores + lax.axis_index("c")
        base = wid * b_per_w
        pltpu.sync_copy(idx_hbm.at[pl.ds(base, b_per_w)], idx_v)
        pltpu.sync_copy(pos_hbm.at[pl.ds(base, b_per_w)], pos_v)
        pltpu.sync_copy(ppos_hbm, ppos_v)

        def body(g, carry):
            vec = idx_v[pl.ds(g * 16, 16)]
            for j in range(16):
                r = vec[j]
                pltpu.async_copy(
                    ptab_hbm.at[pl.ds(r, 1)],
                    rows_v.at[pl.ds(g * 16 + j, 1)],
                    sem,
                )
            return carry

        lax.fori_loop(0, b_per_w // 16, body, 0)
        pltpu.make_async_copy(
            ptab_hbm.at[pl.ds(0, b_per_w)], rows_v, sem
        ).wait()

        def add_body(g, carry):
            pv = pos_v[pl.ds(g * 16, 16)]
            for j in range(16):
                p = pv[j]
                row = rows_v.at[g * 16 + j]
                prow = ppos_v.at[p]
                for c in range(_L64 // 16):
                    s = pl.ds(c * 16, 16)
                    row[s] = row[s] + prow[s]
            return carry

        lax.fori_loop(0, b_per_w // 16, add_body, 0)
        pltpu.sync_copy(rows_v, out_hbm.at[pl.ds(base, b_per_w)])

    return gather_k(p_tab, p_pos, idx, pos)


def kernel(word_ids, prev_pos, word_table, pos_table, W, b):
    B = word_ids.shape[0]
    V = word_table.shape[0]
    W64 = jnp.zeros((_L64, W.shape[1]), W.dtype).at[:_NUM_LABELS].set(W)
    b64 = jnp.zeros((_L64,), b.dtype).at[:_NUM_LABELS].set(b)
    p_tab, p_pos = _project(word_table.T, pos_table, W64, b64, V)
    out64 = _sc_gather_add(p_tab, p_pos, word_ids.astype(jnp.int32),
                           prev_pos.astype(jnp.int32), B)
    return out64[:, :_NUM_LABELS]


# speedup vs baseline: 1.0101x; 1.0101x over previous
import functools

import jax
import jax.numpy as jnp
from jax import lax
from jax.experimental import pallas as pl
from jax.experimental.pallas import tpu as pltpu
from jax.experimental.pallas import tpu_sc as plsc

_NUM_LABELS = 50
_WORD_DIM = 50
_POS_DIM = 15
_L64 = 64


def _proj_body(tabt_ref, W_ref, ptab_ref, b_ref, out_ref, pout_ref):
    xt = tabt_ref[...]
    W64 = W_ref[...]
    Ww = W64[:, :_WORD_DIM]
    scores = lax.dot_general(
        xt.astype(jnp.bfloat16), Ww.astype(jnp.bfloat16),
        (((0,), (1,)), ((), ())), preferred_element_type=jnp.float32)
    out_ref[...] = scores + b_ref[...]

    @pl.when(pl.program_id(0) == 0)
    def _():
        Wp = W64[:, _WORD_DIM:]
        pout_ref[...] = lax.dot_general(ptab_ref[...], Wp,
                                        (((1,), (1,)), ((), ())),
                                        precision=lax.Precision.HIGHEST)


def _project(table_t, pos_table, W64, b64, V):
    blk = 20480
    grid = (pl.cdiv(V, blk),)
    return pl.pallas_call(
        _proj_body,
        grid=grid,
        in_specs=[
            pl.BlockSpec((_WORD_DIM, blk), lambda i: (0, i)),
            pl.BlockSpec((_L64, _WORD_DIM + _POS_DIM), lambda i: (0, 0)),
            pl.BlockSpec((_NUM_LABELS, _POS_DIM), lambda i: (0, 0)),
            pl.BlockSpec((1, _L64), lambda i: (0, 0)),
        ],
        out_specs=[
            pl.BlockSpec((blk, _L64), lambda i: (i, 0)),
            pl.BlockSpec((_NUM_LABELS, _L64), lambda i: (0, 0)),
        ],
        out_shape=[
            jax.ShapeDtypeStruct((V, _L64), jnp.float32),
            jax.ShapeDtypeStruct((_NUM_LABELS, _L64), jnp.float32),
        ],
    )(table_t, W64, pos_table, b64.reshape(1, _L64))


def _sc_gather_add(p_tab, p_pos, idx, pos, B):
    info = plsc.get_sparse_core_info()
    nw = info.num_cores * info.num_subcores
    b_per_w = B // nw
    mesh = plsc.VectorSubcoreMesh(core_axis_name="c", subcore_axis_name="s")

    @functools.partial(
        pl.kernel,
        mesh=mesh,
        out_type=jax.ShapeDtypeStruct((B, _L64), jnp.float32),
        compiler_params=pltpu.CompilerParams(use_tc_tiling_on_sc=True),
        scratch_types=[
            pltpu.VMEM((b_per_w,), jnp.int32),
            pltpu.VMEM((b_per_w,), jnp.int32),
            pltpu.VMEM((b_per_w, _L64), jnp.float32),
            pltpu.VMEM((_NUM_LABELS, _L64), jnp.float32),
            pltpu.SemaphoreType.DMA,
        ],
    )
    def gather_k(ptab_hbm, ppos_hbm, idx_hbm, pos_hbm, out_hbm,
                 idx_v, pos_v, rows_v, ppos_v, sem):
        wid = lax.axis_index("s") * info.num_cores + lax.axis_index("c")
        base = wid * b_per_w
        pltpu.sync_copy(idx_hbm.at[pl.ds(base, b_per_w)], idx_v)
        pltpu.sync_copy(pos_hbm.at[pl.ds(base, b_per_w)], pos_v)
        pltpu.sync_copy(ppos_hbm, ppos_v)

        def body(g, carry):
            vec = idx_v[pl.ds(g * 16, 16)]
            for j in range(16):
                r = vec[j]
                pltpu.async_copy(
                    ptab_hbm.at[pl.ds(r, 1)],
                    rows_v.at[pl.ds(g * 16 + j, 1)],
                    sem,
                )
            return carry

        lax.fori_loop(0, b_per_w // 16, body, 0)
        pltpu.make_async_copy(
            ptab_hbm.at[pl.ds(0, b_per_w)], rows_v, sem
        ).wait()

        def add_body(g, carry):
            pv = pos_v[pl.ds(g * 16, 16)]
            for j in range(16):
                p = pv[j]
                row = rows_v.at[g * 16 + j]
                prow = ppos_v.at[p]
                for c in range(_L64 // 16):
                    s = pl.ds(c * 16, 16)
                    row[s] = row[s] + prow[s]
            return carry

        lax.fori_loop(0, b_per_w // 16, add_body, 0)
        pltpu.sync_copy(rows_v, out_hbm.at[pl.ds(base, b_per_w)])

    return gather_k(p_tab, p_pos, idx, pos)


def kernel(word_ids, prev_pos, word_table, pos_table, W, b):
    B = word_ids.shape[0]
    V = word_table.shape[0]
    W64 = jnp.zeros((_L64, W.shape[1]), W.dtype).at[:_NUM_LABELS].set(W)
    b64 = jnp.zeros((_L64,), b.dtype).at[:_NUM_LABELS].set(b)
    p_tab, p_pos = _project(word_table.T, pos_table, W64, b64, V)
    out64 = _sc_gather_add(p_tab, p_pos, word_ids.astype(jnp.int32),
                           prev_pos.astype(jnp.int32), B)
    return out64[:, :_NUM_LABELS]
